# Initial kernel scaffold; baseline (speedup 1.0000x reference)
import jax
import jax.numpy as jnp
from jax.experimental import pallas as pl


def kernel(x1, x2, w_tx1, lm1, theta1, w_tx2, lm2, theta2, ln1_g, ln1_b, qkv_w, qkv_b, proj_w, proj_b, ln2_g, ln2_b, mlp_w1, mlp_b1, mlp_w2, mlp_b2, fc_w1, fc_b1, fc_w2, fc_b2):
    raise NotImplementedError("write your pallas kernel here")


# speedup vs baseline: 2.1040x; 2.1040x over previous
import numpy as np
import jax
import jax.numpy as jnp
from jax.experimental import pallas as pl
from jax.experimental.pallas import tpu as pltpu

_B, _C, _H, _W = 4, 96, 128, 128
_HW = _H * _W
_NS = 2
_SH = _H // _NS
_SLAB = (_SH + 4) * _W
_CH = 2048
_CHE = _CH + 2 * _W
_CCH = 8


def _dct_mat(n):
    k = np.arange(n)[:, None].astype(np.float64)
    i = np.arange(n)[None, :].astype(np.float64)
    m = np.cos(np.pi * (2 * i + 1) * k / (2 * n)) * np.sqrt(2.0 / n)
    m[0] *= 1.0 / np.sqrt(2.0)
    return m

_DHS = np.ascontiguousarray(_dct_mat(_H) / _C, dtype=np.float32)
_DWT = np.ascontiguousarray(_dct_mat(_W).T, dtype=np.float32)


def _gelu(x):
    ax = jnp.abs(x) * np.float32(0.7071067811865476)
    t = 1.0 / (1.0 + 0.3275911 * ax)
    poly = t * (0.254829592 + t * (-0.284496736 + t * (
        1.421413741 + t * (-1.453152027 + t * 1.061405429))))
    erf_ax = 1.0 - poly * jnp.exp(-ax * ax)
    erf = jnp.where(x < 0, -erf_ax, erf_ax)
    return 0.5 * x * (1.0 + erf)


def _conv_swin_kernel(x1_ref, x2_ref, wcat_ref, wc_ref, bc_ref,
                      g1_ref, b1_ref, g2_ref, b2_ref,
                      w1_ref, bm1_ref, w2_ref, bm2_ref,
                      out_ref, x6_ref):
    lane = jax.lax.broadcasted_iota(jnp.int32, (1, _CH), 1) % _W
    wl = lane != 0
    wr = lane != (_W - 1)

    def ln(t, g, b):
        m = jnp.mean(t, axis=0, keepdims=True)
        d = t - m
        v = jnp.mean(d * d, axis=0, keepdims=True)
        return d * jax.lax.rsqrt(v + 1e-5) * g + b

    for ci in range((_SH * _W) // _CH):
        base = ci * _CH
        for i, xr in enumerate((x1_ref, x2_ref)):
            for kdy in range(3):
                st = base + kdy * _W
                x6_ref[(i * 3 + kdy) * _C:(i * 3 + kdy + 1) * _C, :] = (
                    xr[0, 0, :, st:st + _CHE])
        a = jnp.dot(wcat_ref[...], x6_ref[...],
                    preferred_element_type=jnp.float32)
        am = a[0:_C, _W - 1:_W - 1 + _CH]
        a0 = a[_C:2 * _C, _W:_W + _CH]
        ap = a[2 * _C:3 * _C, _W + 1:_W + 1 + _CH]
        conv = a0 + jnp.where(wl, am, 0.0) + jnp.where(wr, ap, 0.0)
        c1 = x1_ref[0, 0, :, base + 2 * _W:base + 2 * _W + _CH]
        c2 = x2_ref[0, 0, :, base + 2 * _W:base + 2 * _W + _CH]
        t = conv + 2.0 * (c1 + c2)
        tn = ln(t, g1_ref[...], b1_ref[...])
        t = t + jnp.dot(wc_ref[...], tn,
                        preferred_element_type=jnp.float32) + bc_ref[...]
        tn2 = ln(t, g2_ref[...], b2_ref[...])
        h = jnp.dot(w1_ref[...], tn2,
                    preferred_element_type=jnp.float32) + bm1_ref[...]
        h = _gelu(h)
        t = t + jnp.dot(w2_ref[...], h,
                        preferred_element_type=jnp.float32) + bm2_ref[...]
        out_ref[0, :, base:base + _CH] = t


def _dct_kernel(x_ref, dwt_ref, dhs_ref, e_ref, z_ref):
    c = pl.program_id(1)

    @pl.when(c == 0)
    def _():
        e_ref[...] = jnp.zeros_like(e_ref)

    x = x_ref[0]
    y = jnp.dot(x.reshape(_CCH * _H, _W), dwt_ref[...],
                preferred_element_type=jnp.float32)
    for i in range(_CCH):
        z_ref[:, i * _W:(i + 1) * _W] = y[i * _H:(i + 1) * _H, :]
    p = jnp.dot(dhs_ref[...], z_ref[...],
                preferred_element_type=jnp.float32)
    acc = jnp.abs(p[:, 0:_W])
    for i in range(1, _CCH):
        acc = acc + jnp.abs(p[:, i * _W:(i + 1) * _W])
    e_ref[...] = e_ref[...] + acc[None, :, :]


def _topk_kernel(e_ref, f1_ref, b1_ref, f2_ref, b2_ref, att_ref):
    e0 = e_ref[0]
    fi = (jax.lax.broadcasted_iota(jnp.int32, (_H, _W), 0) * _W
          + jax.lax.broadcasted_iota(jnp.int32, (_H, _W), 1))
    li = jax.lax.broadcasted_iota(jnp.int32, (1, _W), 1)

    def body(i, carry):
        e, tk = carry
        cm = jnp.max(e, axis=0, keepdims=True)
        m = jnp.max(cm, axis=1, keepdims=True)
        sel = jnp.where(e == m, fi, _HW)
        cmn = jnp.min(sel, axis=0, keepdims=True)
        idx = jnp.min(cmn, axis=1, keepdims=True)
        e = jnp.where(fi == idx, -1.0, e)
        tk = jnp.where(li == i, m, tk)
        return e, tk

    _, tk = jax.lax.fori_loop(0, _C, body,
                              (e0, jnp.zeros((1, _W), jnp.float32)))
    z1 = jnp.dot(tk, f1_ref[...], preferred_element_type=jnp.float32)
    z1 = jnp.maximum(z1 + b1_ref[...], 0.0)
    z2 = jnp.sum(z1 * f2_ref[...], axis=1, keepdims=True) + b2_ref[0:1, 0:1]
    att_ref[...] = jnp.broadcast_to(jax.nn.sigmoid(z2), (1, 1, _W))


def _final_kernel(att_ref, x1_ref, x2_ref, cr_ref, o1_ref, o2_ref):
    s = 1.0 + att_ref[pl.program_id(0)]
    cs = cr_ref[...] * s
    o1_ref[...] = x1_ref[...] + cs
    o2_ref[...] = x2_ref[...] + cs


def kernel(x1, x2, w_tx1, lm1, theta1, w_tx2, lm2, theta2,
           ln1_g, ln1_b, qkv_w, qkv_b, proj_w, proj_b,
           ln2_g, ln2_b, mlp_w1, mlp_b1, mlp_w2, mlp_b2,
           fc_w1, fc_b1, fc_w2, fc_b2):
    f32 = jnp.float32

    def eff(w, lm, th):
        delta = th[0] * lm * w.sum((2, 3))
        return w.at[:, :, 1, 1].add(-delta * w[:, :, 1, 1])

    w1e = eff(w_tx1, lm1, theta1)
    w2e = eff(w_tx2, lm2, theta2)
    wcat = jnp.concatenate([
        jnp.concatenate([we[:, :, dy, dx] for we in (w1e, w2e)
                         for dy in range(3)], axis=1)
        for dx in range(3)], axis=0)

    wv = qkv_w[2 * _C:]
    bv = qkv_b[2 * _C:]
    wc = proj_w @ wv
    bc = proj_w @ bv + proj_b

    tile = lambda v: jnp.broadcast_to(v[:, None], (v.shape[0], _CH)).astype(f32)
    bc_t = tile(bc)
    g1_t, b1_t = tile(ln1_g), tile(ln1_b)
    g2_t, b2_t = tile(ln2_g), tile(ln2_b)
    bm1_t, bm2_t = tile(mlp_b1), tile(mlp_b2)

    def mkslab(x):
        xp = jnp.pad(x, ((0, 0), (0, 0), (2, 2), (0, 0)))
        sl = jnp.stack([xp[:, :, _SH * s:_SH * s + _SH + 4, :]
                        for s in range(_NS)], axis=1)
        return sl.reshape(_B, _NS, _C, _SLAB)

    x1s = mkslab(x1)
    x2s = mkslab(x2)

    wspec = lambda shape: pl.BlockSpec(shape, lambda b, s: (0, 0))
    cross = pl.pallas_call(
        _conv_swin_kernel,
        grid=(_B, _NS),
        in_specs=[
            pl.BlockSpec((1, 1, _C, _SLAB), lambda b, s: (b, s, 0, 0)),
            pl.BlockSpec((1, 1, _C, _SLAB), lambda b, s: (b, s, 0, 0)),
            wspec((3 * _C, 6 * _C)),
            wspec((_C, _C)), wspec((_C, _CH)),
            wspec((_C, _CH)), wspec((_C, _CH)),
            wspec((_C, _CH)), wspec((_C, _CH)),
            wspec((4 * _C, _C)), wspec((4 * _C, _CH)),
            wspec((_C, 4 * _C)), wspec((_C, _CH)),
        ],
        out_specs=pl.BlockSpec((1, _C, _SH * _W), lambda b, s: (b, 0, s)),
        out_shape=jax.ShapeDtypeStruct((_B, _C, _HW), f32),
        scratch_shapes=[pltpu.VMEM((6 * _C, _CHE), f32)],
        compiler_params=pltpu.CompilerParams(
            dimension_semantics=("parallel", "parallel"),
            vmem_limit_bytes=100 * 1024 * 1024),
    )(x1s, x2s, wcat, wc, bc_t, g1_t, b1_t, g2_t, b2_t,
      mlp_w1, bm1_t, mlp_w2, bm2_t)

    cross4 = cross.reshape(_B, _C, _H, _W)
    energy = pl.pallas_call(
        _dct_kernel,
        grid=(_B, _C // _CCH),
        in_specs=[
            pl.BlockSpec((1, _CCH, _H, _W), lambda b, c: (b, c, 0, 0)),
            pl.BlockSpec((_W, _W), lambda b, c: (0, 0)),
            pl.BlockSpec((_H, _H), lambda b, c: (0, 0)),
        ],
        out_specs=pl.BlockSpec((1, _H, _W), lambda b, c: (b, 0, 0)),
        out_shape=jax.ShapeDtypeStruct((_B, _H, _W), f32),
        scratch_shapes=[pltpu.VMEM((_H, _CCH * _W), f32)],
        compiler_params=pltpu.CompilerParams(
            dimension_semantics=("parallel", "arbitrary"),
            vmem_limit_bytes=100 * 1024 * 1024),
    )(cross4, jnp.asarray(_DWT), jnp.asarray(_DHS))

    f1p = jnp.zeros((_W, _W), f32).at[0:_C, 0:_C // 4].set(fc_w1.T)
    b1p = jnp.zeros((1, _W), f32).at[0, 0:_C // 4].set(fc_b1)
    f2p = jnp.zeros((1, _W), f32).at[0, 0:_C // 4].set(fc_w2[0])
    b2p = jnp.broadcast_to(fc_b2.reshape(1, 1), (1, _W)).astype(f32)
    att = pl.pallas_call(
        _topk_kernel,
        grid=(_B,),
        in_specs=[
            pl.BlockSpec((1, _H, _W), lambda b: (b, 0, 0)),
            pl.BlockSpec((_W, _W), lambda b: (0, 0)),
            pl.BlockSpec((1, _W), lambda b: (0, 0)),
            pl.BlockSpec((1, _W), lambda b: (0, 0)),
            pl.BlockSpec((1, _W), lambda b: (0, 0)),
        ],
        out_specs=pl.BlockSpec((1, 1, _W), lambda b: (b, 0, 0)),
        out_shape=jax.ShapeDtypeStruct((_B, 1, _W), f32),
        compiler_params=pltpu.CompilerParams(
            dimension_semantics=("parallel",)),
    )(energy, f1p, b1p, f2p, b2p)
    attv = att[:, 0, 0]

    nsp = 4
    sch = _HW // nsp
    x1f = x1.reshape(_B, _C, _HW)
    x2f = x2.reshape(_B, _C, _HW)
    dspec = lambda: pl.BlockSpec((1, _C, sch), lambda b, j: (b, 0, j))
    out1, out2 = pl.pallas_call(
        _final_kernel,
        grid=(_B, nsp),
        in_specs=[
            pl.BlockSpec(memory_space=pltpu.SMEM),
            dspec(), dspec(), dspec(),
        ],
        out_specs=(dspec(), dspec()),
        out_shape=(jax.ShapeDtypeStruct((_B, _C, _HW), f32),
                   jax.ShapeDtypeStruct((_B, _C, _HW), f32)),
        compiler_params=pltpu.CompilerParams(
            dimension_semantics=("parallel", "parallel")),
    )(attv, x1f, x2f, cross)
    return (out1.reshape(_B, _C, _H, _W), out2.reshape(_B, _C, _H, _W))
